# baseline (speedup 1.0000x reference)
import functools

import jax
import jax.numpy as jnp
from jax.experimental import pallas as pl
from jax.experimental.pallas import tpu as pltpu

_NH = 16
_HD = 64
_RATIO = 0.2
_RB = 256
_AUG = 128


def _pow2i(e):
    return jax.lax.bitcast_convert_type((e + 127) << 23, jnp.float32)


def _expo(ax):
    bits = jax.lax.bitcast_convert_type(ax, jnp.int32)
    return ((bits >> 23) & 0xFF) - 127


def _quant(x):
    maxval = jnp.float32(448.0)
    xc = jnp.clip(x, -maxval, maxval)
    ax = jnp.abs(xc)
    e = jnp.clip(_expo(jnp.maximum(ax, jnp.float32(2.0 ** -9))), -6, 8)
    mq = jnp.round(xc * _pow2i(3 - e))
    return mq * _pow2i(e - 3)


def _level_val(c):
    a = jnp.abs(c)
    e6 = jnp.maximum((a - 8) >> 3, 0)
    m = (a - (e6 << 3)).astype(jnp.float32)
    v = m * _pow2i(e6 - 9)
    return jnp.where(c < 0, -v, v)


def _quant_pos(y):
    xc = jnp.minimum(y, jnp.float32(448.0))
    e = jnp.clip(_expo(jnp.maximum(xc, jnp.float32(2.0 ** -9))), -6, 8)
    return jnp.round(xc * _pow2i(3 - e)) * _pow2i(e - 3)


def _f16r(x):
    ax = jnp.abs(x)
    e = jnp.clip(_expo(jnp.maximum(ax, jnp.float32(2.0 ** -30))), -14, 15)
    y = jnp.round(x * _pow2i(10 - e)) * _pow2i(e - 10)
    return jnp.where(jnp.abs(y) > 65504.0, jnp.sign(x) * jnp.inf, y)


def _pass1_kernel(aw_ref, v_ref, e_ref, s_ref, m_ref, thr_ref, vq_scr, *, kk):
    i = pl.program_id(1)

    @pl.when(i == 0)
    def _():
        vq = _quant(v_ref[0])
        tl = vq.shape[0]
        pad = jnp.concatenate(
            [jnp.ones((tl, 1), jnp.float32), jnp.zeros((tl, _AUG - _HD - 1), jnp.float32)],
            axis=1)
        vq_scr[...] = jnp.concatenate([vq, pad], axis=1).astype(jnp.bfloat16)
        thr_ref[...] = jnp.zeros_like(thr_ref)

    x = aw_ref[0]
    q = _quant(x)

    rb = q.shape[0]
    tl = q.shape[1]
    lo = jnp.full((rb, 1), -126, jnp.int32)
    hi = jnp.full((rb, 1), 127, jnp.int32)
    cnt_hi = jnp.zeros((rb, 1), jnp.float32)
    for _ in range(4):
        w = hi - lo
        m1 = lo + (w >> 2)
        m2 = lo + (w >> 1)
        m3 = lo + ((3 * w) >> 2)
        v1, v2, v3 = _level_val(m1), _level_val(m2), _level_val(m3)
        c1 = jnp.sum((q >= v1).astype(jnp.float32), axis=1, keepdims=True)
        c2 = jnp.sum((q >= v2).astype(jnp.float32), axis=1, keepdims=True)
        c3 = jnp.sum((q >= v3).astype(jnp.float32), axis=1, keepdims=True)
        t1, t2, t3 = c1 >= kk, c2 >= kk, c3 >= kk
        lo = jnp.where(t3, m3, jnp.where(t2, m2, jnp.where(t1, m1, lo)))
        hi = jnp.where(t1, jnp.where(t2, jnp.where(t3, hi, m3), m2), m1)
        cnt_hi = jnp.where(t1, jnp.where(t2, jnp.where(t3, cnt_hi, c3), c2), c1)
    cth_val = _level_val(lo)

    gt = q > cth_val
    is_tie = q == cth_val
    limit = kk - cnt_hi
    ii = jax.lax.broadcasted_iota(jnp.int32, (128, 128), 0)
    jj = jax.lax.broadcasted_iota(jnp.int32, (128, 128), 1)
    tri = (ii <= jj).astype(jnp.bfloat16)
    tie_f = is_tie.astype(jnp.bfloat16)
    parts = []
    running = jnp.zeros((rb, 1), jnp.float32)
    for c in range(tl // 128):
        local = jnp.dot(tie_f[:, c * 128:(c + 1) * 128], tri,
                        preferred_element_type=jnp.float32)
        parts.append(local + running)
        running = running + local[:, 127:128]
    rank = jnp.concatenate(parts, axis=1)

    mask = gt | (is_tie & (rank <= limit))
    thr_ref[...] = thr_ref[...] + jnp.sum(cth_val)

    dense_exp = jnp.where(mask, 0.0, _quant_pos(jnp.exp(q))).astype(jnp.bfloat16)
    sparse_w = jnp.where(mask, q, 0.0).astype(jnp.bfloat16)
    maskb = mask.astype(jnp.bfloat16)

    vq = vq_scr[...]
    mm = functools.partial(jnp.dot, preferred_element_type=jnp.float32)
    em = mm(dense_exp, vq)
    sm = mm(sparse_w, vq)
    mmk = mm(maskb, vq)
    sq = jnp.concatenate([_quant(sm[:, :_HD]), sm[:, _HD:]], axis=1)

    e_ref[0] = em
    s_ref[0] = sq
    m_ref[0] = mmk


def _pass2_kernel(thr_ref, e_ref, s_ref, m_ref, o_ref, *, nrows):
    mean_thr = thr_ref[0][:, 0:1] / nrows
    kf = jnp.exp(mean_thr)
    bf = (1.0 - mean_thr) * kf

    em = e_ref[0]
    sq = s_ref[0]
    mmk = m_ref[0]

    dense = _f16r(em[:, :_HD])
    s1 = _f16r(kf * sq[:, :_HD])
    s2 = _f16r(bf * mmk[:, :_HD])
    sparse = _f16r(s1 + s2)
    total = _f16r(dense + sparse)
    den = _f16r(em[:, _HD:_HD + 1] + kf * sq[:, _HD:_HD + 1]
                + bf * mmk[:, _HD:_HD + 1])
    out = _f16r(total / den)
    o_ref[0] = _quant(out)


def kernel(attn_weights, v):
    nh, sl, tl = attn_weights.shape
    kk = int(tl * _RATIO)
    rb = min(_RB, sl)
    nb = sl // rb

    em, sq, mmk, thr = pl.pallas_call(
        functools.partial(_pass1_kernel, kk=kk),
        grid=(nh, nb),
        in_specs=[
            pl.BlockSpec((1, rb, tl), lambda h, i: (h, i, 0)),
            pl.BlockSpec((1, tl, _HD), lambda h, i: (h, 0, 0)),
        ],
        out_specs=[
            pl.BlockSpec((1, rb, _AUG), lambda h, i: (h, i, 0)),
            pl.BlockSpec((1, rb, _AUG), lambda h, i: (h, i, 0)),
            pl.BlockSpec((1, rb, _AUG), lambda h, i: (h, i, 0)),
            pl.BlockSpec((1, 1, 128), lambda h, i: (h, 0, 0)),
        ],
        out_shape=[
            jax.ShapeDtypeStruct((nh, sl, _AUG), jnp.float32),
            jax.ShapeDtypeStruct((nh, sl, _AUG), jnp.float32),
            jax.ShapeDtypeStruct((nh, sl, _AUG), jnp.float32),
            jax.ShapeDtypeStruct((nh, 1, 128), jnp.float32),
        ],
        scratch_shapes=[pltpu.VMEM((tl, _AUG), jnp.bfloat16)],
    )(attn_weights, v)

    out = pl.pallas_call(
        functools.partial(_pass2_kernel, nrows=float(sl)),
        grid=(nh, nb),
        in_specs=[
            pl.BlockSpec((1, 1, 128), lambda h, i: (h, 0, 0)),
            pl.BlockSpec((1, rb, _AUG), lambda h, i: (h, i, 0)),
            pl.BlockSpec((1, rb, _AUG), lambda h, i: (h, i, 0)),
            pl.BlockSpec((1, rb, _AUG), lambda h, i: (h, i, 0)),
        ],
        out_specs=pl.BlockSpec((1, rb, _HD), lambda h, i: (h, i, 0)),
        out_shape=jax.ShapeDtypeStruct((nh, sl, _HD), jnp.float32),
    )(thr, em, sq, mmk)
    return out.astype(jnp.float16)


# speedup vs baseline: 1.3211x; 1.3211x over previous
import functools

import jax
import jax.numpy as jnp
from jax.experimental import pallas as pl
from jax.experimental.pallas import tpu as pltpu

_NH = 16
_HD = 64
_RATIO = 0.2
_RB = 256
_AUG = 128


def _pow2i(e):
    return jax.lax.bitcast_convert_type((e + 127) << 23, jnp.float32)


def _expo(ax):
    bits = jax.lax.bitcast_convert_type(ax, jnp.int32)
    return ((bits >> 23) & 0xFF) - 127


def _quant(x):
    xc = jnp.clip(x, jnp.float32(-448.0), jnp.float32(448.0))
    bits = jax.lax.bitcast_convert_type(xc, jnp.int32)
    tie = (bits >> 20) & 1
    rbits = (bits + (524287 + tie)) & jnp.int32(-1048576)
    ybig = jax.lax.bitcast_convert_type(rbits, jnp.float32)
    c = jnp.float32(1.5 * 2.0 ** 14)
    ysmall = (xc + c) - c
    return jnp.where(jnp.abs(xc) < jnp.float32(2.0 ** -6), ysmall, ybig)


def _level_val(c):
    a = jnp.abs(c)
    e6 = jnp.maximum((a - 8) >> 3, 0)
    m = (a - (e6 << 3)).astype(jnp.float32)
    v = m * _pow2i(e6 - 9)
    return jnp.where(c < 0, -v, v)


def _quant_pos(y):
    xc = jnp.minimum(y, jnp.float32(448.0))
    bits = jax.lax.bitcast_convert_type(xc, jnp.int32)
    tie = (bits >> 20) & 1
    rbits = (bits + (524287 + tie)) & jnp.int32(-1048576)
    ybig = jax.lax.bitcast_convert_type(rbits, jnp.float32)
    c = jnp.float32(1.5 * 2.0 ** 14)
    ysmall = (xc + c) - c
    return jnp.where(xc < jnp.float32(2.0 ** -6), ysmall, ybig)


def _f16r(x):
    ax = jnp.abs(x)
    e = jnp.clip(_expo(jnp.maximum(ax, jnp.float32(2.0 ** -30))), -14, 15)
    y = jnp.round(x * _pow2i(10 - e)) * _pow2i(e - 10)
    return jnp.where(jnp.abs(y) > 65504.0, jnp.sign(x) * jnp.inf, y)


def _pass1_kernel(aw_ref, v_ref, e_ref, s_ref, m_ref, thr_ref, vq_scr, *, kk):
    i = pl.program_id(1)

    @pl.when(i == 0)
    def _():
        vq = _quant(v_ref[0])
        tl = vq.shape[0]
        pad = jnp.concatenate(
            [jnp.ones((tl, 1), jnp.float32), jnp.zeros((tl, _AUG - _HD - 1), jnp.float32)],
            axis=1)
        vq_scr[...] = jnp.concatenate([vq, pad], axis=1).astype(jnp.bfloat16)
        thr_ref[...] = jnp.zeros_like(thr_ref)

    x = aw_ref[0]
    q = _quant(x)

    rb = q.shape[0]
    tl = q.shape[1]
    lo = jnp.full((rb, 1), -126, jnp.int32)
    hi = jnp.full((rb, 1), 127, jnp.int32)
    cnt_hi = jnp.zeros((rb, 1), jnp.float32)
    for _ in range(8):
        mid = (lo + hi) >> 1
        vmid = _level_val(mid)
        cnt = jnp.sum((q >= vmid).astype(jnp.float32), axis=1, keepdims=True)
        take = cnt >= kk
        lo = jnp.where(take, mid, lo)
        hi = jnp.where(take, hi, mid)
        cnt_hi = jnp.where(take, cnt_hi, cnt)
    cth_val = _level_val(lo)

    gt = q > cth_val
    is_tie = q == cth_val
    limit = kk - cnt_hi
    ii = jax.lax.broadcasted_iota(jnp.int32, (128, 128), 0)
    jj = jax.lax.broadcasted_iota(jnp.int32, (128, 128), 1)
    tri = (ii <= jj).astype(jnp.bfloat16)
    tie_f = is_tie.astype(jnp.bfloat16)
    parts = []
    running = jnp.zeros((rb, 1), jnp.float32)
    for c in range(tl // 128):
        local = jnp.dot(tie_f[:, c * 128:(c + 1) * 128], tri,
                        preferred_element_type=jnp.float32)
        parts.append(local + running)
        running = running + local[:, 127:128]
    rank = jnp.concatenate(parts, axis=1)

    mask = gt | (is_tie & (rank <= limit))
    thr_ref[...] = thr_ref[...] + jnp.sum(cth_val)

    dense_exp = jnp.where(mask, 0.0, _quant_pos(jnp.exp(q))).astype(jnp.bfloat16)
    sparse_w = jnp.where(mask, q, 0.0).astype(jnp.bfloat16)
    maskb = mask.astype(jnp.bfloat16)

    vq = vq_scr[...]
    mm = functools.partial(jnp.dot, preferred_element_type=jnp.float32)
    em = mm(dense_exp, vq)
    sm = mm(sparse_w, vq)
    mmk = mm(maskb, vq)
    sq = jnp.concatenate([_quant(sm[:, :_HD]), sm[:, _HD:]], axis=1)

    e_ref[0] = em
    s_ref[0] = sq
    m_ref[0] = mmk


def _pass2_kernel(thr_ref, e_ref, s_ref, m_ref, o_ref, *, nrows):
    mean_thr = thr_ref[0][:, 0:1] / nrows
    kf = jnp.exp(mean_thr)
    bf = (1.0 - mean_thr) * kf

    em = e_ref[0]
    sq = s_ref[0]
    mmk = m_ref[0]

    dense = _f16r(em[:, :_HD])
    s1 = _f16r(kf * sq[:, :_HD])
    s2 = _f16r(bf * mmk[:, :_HD])
    sparse = _f16r(s1 + s2)
    total = _f16r(dense + sparse)
    den = _f16r(em[:, _HD:_HD + 1] + kf * sq[:, _HD:_HD + 1]
                + bf * mmk[:, _HD:_HD + 1])
    out = _f16r(total / den)
    o_ref[0] = _quant(out)


def kernel(attn_weights, v):
    nh, sl, tl = attn_weights.shape
    kk = int(tl * _RATIO)
    rb = min(_RB, sl)
    nb = sl // rb

    em, sq, mmk, thr = pl.pallas_call(
        functools.partial(_pass1_kernel, kk=kk),
        grid=(nh, nb),
        in_specs=[
            pl.BlockSpec((1, rb, tl), lambda h, i: (h, i, 0)),
            pl.BlockSpec((1, tl, _HD), lambda h, i: (h, 0, 0)),
        ],
        out_specs=[
            pl.BlockSpec((1, rb, _AUG), lambda h, i: (h, i, 0)),
            pl.BlockSpec((1, rb, _AUG), lambda h, i: (h, i, 0)),
            pl.BlockSpec((1, rb, _AUG), lambda h, i: (h, i, 0)),
            pl.BlockSpec((1, 1, 128), lambda h, i: (h, 0, 0)),
        ],
        out_shape=[
            jax.ShapeDtypeStruct((nh, sl, _AUG), jnp.float32),
            jax.ShapeDtypeStruct((nh, sl, _AUG), jnp.float32),
            jax.ShapeDtypeStruct((nh, sl, _AUG), jnp.float32),
            jax.ShapeDtypeStruct((nh, 1, 128), jnp.float32),
        ],
        scratch_shapes=[pltpu.VMEM((tl, _AUG), jnp.bfloat16)],
    )(attn_weights, v)

    out = pl.pallas_call(
        functools.partial(_pass2_kernel, nrows=float(sl)),
        grid=(nh, nb),
        in_specs=[
            pl.BlockSpec((1, 1, 128), lambda h, i: (h, 0, 0)),
            pl.BlockSpec((1, rb, _AUG), lambda h, i: (h, i, 0)),
            pl.BlockSpec((1, rb, _AUG), lambda h, i: (h, i, 0)),
            pl.BlockSpec((1, rb, _AUG), lambda h, i: (h, i, 0)),
        ],
        out_specs=pl.BlockSpec((1, rb, _HD), lambda h, i: (h, i, 0)),
        out_shape=jax.ShapeDtypeStruct((nh, sl, _HD), jnp.float32),
    )(thr, em, sq, mmk)
    return out.astype(jnp.float16)
